# baseline (speedup 1.0000x reference)
import functools

import jax
import jax.numpy as jnp
from jax import lax
from jax.experimental import pallas as pl
from jax.experimental.pallas import tpu as pltpu
from jax.experimental.pallas import tpu_sc as plsc

_F32 = jnp.float32
_I32 = jnp.int32


def _splat(x):
    return jnp.full((16,), x, _I32)


def kernel(indices, table):
    B, H = indices.shape
    V, D = table.shape

    info = plsc.get_sparse_core_info()
    NC, NS = info.num_cores, info.num_subcores
    NW = NC * NS

    mesh = plsc.VectorSubcoreMesh(core_axis_name="c", subcore_axis_name="s")

    BE = 256
    n_full = V // BE
    tail = V - n_full * BE
    K = n_full // NW + 1
    KP = (K + 1) // 2 * 2

    @functools.partial(
        pl.kernel,
        mesh=mesh,
        out_type=jax.ShapeDtypeStruct((V * D,), _F32),
        scratch_types=[
            [pltpu.VMEM((D, BE), _F32)] * 2,
            [pltpu.VMEM((BE * D,), _F32)] * 2,
            pltpu.VMEM((D, tail), _F32),
            pltpu.VMEM((tail * D,), _F32),
            [pltpu.SemaphoreType.DMA] * 2,
            [pltpu.SemaphoreType.DMA] * 2,
        ],
        compiler_params=pltpu.CompilerParams(needs_layout_passes=False),
    )
    def transpose_kernel(tabT, tabR, S, T, St, Tt, isems, osems):
        wid = lax.axis_index("s") * NC + lax.axis_index("c")
        iota64 = lax.iota(_I32, 16) * D

        def strip_of(k):
            return wid + k * NW

        def fire_in(k, b):
            s = strip_of(k)

            @pl.when(s < n_full)
            def _():
                pltpu.async_copy(tabT.at[:, pl.ds(s * BE, BE)], S[b], isems[b])

        def wait_in(b):
            pltpu.make_async_copy(tabT.at[:, pl.ds(0, BE)], S[b], isems[b]).wait()

        def fire_out(k, b):
            s = strip_of(k)
            pltpu.async_copy(T[b], tabR.at[pl.ds(s * BE * D, BE * D)], osems[b])

        def wait_out(b):
            pltpu.make_async_copy(T[b], tabR.at[pl.ds(0, BE * D)], osems[b]).wait()

        def transpose_strip(src, dst, width):
            def cbody(c, carry):
                for g in range(width // 16):
                    vec = src[c, pl.ds(g * 16, 16)]
                    idxv = iota64 + _splat(g * 16 * D + c)
                    plsc.store_scatter(dst, [idxv], vec)
                return carry

            lax.fori_loop(0, D, cbody, 0)

        fire_in(0, 0)

        def body(k2, carry):
            for b in range(2):
                k = k2 * 2 + b
                s = strip_of(k)

                @pl.when(s < n_full)
                def _():
                    wait_in(b)
                    fire_in(k + 1, 1 - b)

                    @pl.when(k >= 2)
                    def _():
                        wait_out(b)

                    transpose_strip(S[b], T[b], BE)
                    fire_out(k, b)

            return carry

        lax.fori_loop(0, KP // 2, body, 0)

        wait_out(0)
        wait_out(1)

        @pl.when(wid == NW - 1)
        def _():
            pltpu.sync_copy(tabT.at[:, pl.ds(n_full * BE, tail)], St)
            transpose_strip(St, Tt, tail)
            pltpu.sync_copy(Tt, tabR.at[pl.ds(n_full * BE * D, tail * D)])

    SUB = 128
    n_hb = B // SUB
    TT = H

    @functools.partial(
        pl.kernel,
        mesh=mesh,
        out_type=jax.ShapeDtypeStruct((H, D, B), _F32),
        scratch_types=[
            [pltpu.VMEM((SUB,), _I32)] * 2,
            [pltpu.VMEM((SUB,), _I32)] * 2,
            [pltpu.VMEM((SUB, SUB), _F32)] * 2,
            [pltpu.VMEM((D, SUB), _F32)] * 2,
            [pltpu.SemaphoreType.DMA] * 2,
            [pltpu.SemaphoreType.DMA] * 2,
            [pltpu.SemaphoreType.DMA] * 2,
        ],
        compiler_params=pltpu.CompilerParams(needs_layout_passes=False),
    )
    def gather_kernel(idxT, tabR2, outP, idxb, jb, R, OB, isems, gsems, osems):
        wid = lax.axis_index("s") * NC + lax.axis_index("c")
        b0 = wid * SUB
        iota = lax.iota(_I32, 16)

        def fire_idx(t, b):
            pltpu.async_copy(idxT.at[t, pl.ds(b0, SUB)], idxb[b], isems[b])

        def wait_idx(b):
            pltpu.make_async_copy(
                idxT.at[0, pl.ds(0, SUB)], idxb[b], isems[b]
            ).wait()

        def prep_and_fire_gather(b):
            for g in range(SUB // 16):
                ev = idxb[b][pl.ds(g * 16, 16)]
                jb[b][pl.ds(g * 16, 16)] = lax.shift_right_logical(ev, 1)
            pltpu.async_copy(tabR2.at[jb[b]], R[b], gsems[b])

        def wait_gather(b):
            pltpu.make_async_copy(
                tabR2.at[pl.ds(0, SUB)], R[b], gsems[b]
            ).wait()

        def fire_out(t, b):
            pltpu.async_copy(OB[b], outP.at[t, :, pl.ds(b0, SUB)], osems[b])

        def wait_out(b):
            pltpu.make_async_copy(
                OB[b], outP.at[0, :, pl.ds(0, SUB)], osems[b]
            ).wait()

        def transpose_block(b):
            for g in range(SUB // 16):
                rvec = _splat(g * 16) + iota
                ev = idxb[b][pl.ds(g * 16, 16)]
                par64 = lax.shift_left(jnp.bitwise_and(ev, 1), 6)

                def cbody(c, carry):
                    cvec = par64 + _splat(c)
                    vec = plsc.load_gather(R[b], [rvec, cvec])
                    OB[b][c, pl.ds(g * 16, 16)] = vec
                    return carry

                lax.fori_loop(0, D, cbody, 0)

        fire_idx(0, 0)
        wait_idx(0)
        prep_and_fire_gather(0)
        fire_idx(1, 1)

        def body(t2, carry):
            for b in range(2):
                t = t2 * 2 + b

                @pl.when(t < TT)
                def _():
                    wait_gather(b)

                    @pl.when(t + 1 < TT)
                    def _():
                        wait_idx(1 - b)
                        prep_and_fire_gather(1 - b)

                    @pl.when(t >= 2)
                    def _():
                        wait_out(b)

                    transpose_block(b)
                    fire_out(t, b)

                    @pl.when(t + 2 < TT)
                    def _():
                        fire_idx(t + 2, b)

            return carry

        lax.fori_loop(0, TT // 2, body, 0)
        wait_out(0)
        wait_out(1)

    tabT = table.T
    idxT = indices.T
    tabR = transpose_kernel(tabT)
    tabR2 = tabR.reshape(V // 2, 2 * D)
    outP = gather_kernel(idxT, tabR2)
    return outP.transpose(2, 0, 1)


# speedup vs baseline: 1.9709x; 1.9709x over previous
import functools

import jax
import jax.numpy as jnp
from jax import lax
from jax.experimental import pallas as pl
from jax.experimental.pallas import tpu as pltpu
from jax.experimental.pallas import tpu_sc as plsc

_F32 = jnp.float32
_I32 = jnp.int32


def _splat(x):
    return jnp.full((16,), x, _I32)


def kernel(indices, table):
    B, H = indices.shape
    V, D = table.shape

    info = plsc.get_sparse_core_info()
    NC, NS = info.num_cores, info.num_subcores
    NW = NC * NS

    mesh = plsc.VectorSubcoreMesh(core_axis_name="c", subcore_axis_name="s")

    BE = 256
    n_full = V // BE
    tail = V - n_full * BE
    K = n_full // NW + 1
    KP = (K + 1) // 2 * 2

    @functools.partial(
        pl.kernel,
        mesh=mesh,
        out_type=jax.ShapeDtypeStruct((V * D,), _F32),
        scratch_types=[
            [pltpu.VMEM((D, BE), _F32)] * 2,
            [pltpu.VMEM((BE * D,), _F32)] * 2,
            pltpu.VMEM((D, tail), _F32),
            pltpu.VMEM((tail * D,), _F32),
            [pltpu.SemaphoreType.DMA] * 2,
            [pltpu.SemaphoreType.DMA] * 2,
        ],
        compiler_params=pltpu.CompilerParams(needs_layout_passes=False),
    )
    def transpose_kernel(tabT, tabTail, tabR, S, T, St, Tt, isems, osems):
        wid = lax.axis_index("s") * NC + lax.axis_index("c")
        iota = lax.iota(_I32, 16)

        def strip_of(k):
            return wid + k * NW

        def fire_in(k, b):
            s = strip_of(k)

            @pl.when(s < n_full)
            def _():
                pltpu.async_copy(tabT.at[:, pl.ds(s * BE, BE)], S[b], isems[b])

        def wait_in(b):
            pltpu.make_async_copy(tabT.at[:, pl.ds(0, BE)], S[b], isems[b]).wait()

        def fire_out(k, b):
            s = strip_of(k)
            pltpu.async_copy(T[b], tabR.at[pl.ds(s * BE * D, BE * D)], osems[b])

        def wait_out(b):
            pltpu.make_async_copy(T[b], tabR.at[pl.ds(0, BE * D)], osems[b]).wait()

        def transpose_strip(src, dst, width):
            def ibody(ib, carry):
                i0 = ib * 16
                iv = iota + _splat(i0)
                sv = _splat(i0 * D)
                for cb in range(D // 16):
                    c0 = cb * 16
                    for d in range(16):
                        dia = jnp.bitwise_and(iota + d, 15)
                        cvec = dia + c0
                        vec = plsc.load_gather(src, [cvec, iv])
                        plsc.store_scatter(dst, [sv + (iota * D + cvec)], vec)
                return carry

            lax.fori_loop(0, width // 16, ibody, 0)

        fire_in(0, 0)

        def body(k2, carry):
            for b in range(2):
                k = k2 * 2 + b
                s = strip_of(k)

                @pl.when(s < n_full)
                def _():
                    wait_in(b)
                    fire_in(k + 1, 1 - b)

                    @pl.when(k >= 2)
                    def _():
                        wait_out(b)

                    transpose_strip(S[b], T[b], BE)
                    fire_out(k, b)

            return carry

        lax.fori_loop(0, KP // 2, body, 0)

        wait_out(0)
        wait_out(1)

        @pl.when(wid == NW - 1)
        def _():
            pltpu.sync_copy(tabTail, St)
            transpose_strip(St, Tt, tail)
            pltpu.sync_copy(Tt, tabR.at[pl.ds(n_full * BE * D, tail * D)])

    SUB = 128
    TT = H

    @functools.partial(
        pl.kernel,
        mesh=mesh,
        out_type=jax.ShapeDtypeStruct((H, D, B), _F32),
        scratch_types=[
            [pltpu.VMEM((SUB,), _I32)] * 2,
            [pltpu.VMEM((SUB,), _I32)] * 2,
            [pltpu.VMEM((SUB, 2 * D), _F32)] * 2,
            [pltpu.VMEM((D, SUB), _F32)] * 2,
            [pltpu.SemaphoreType.DMA] * 2,
            [pltpu.SemaphoreType.DMA] * 2,
            [pltpu.SemaphoreType.DMA] * 2,
        ],
        compiler_params=pltpu.CompilerParams(needs_layout_passes=False),
    )
    def gather_kernel(idxT, tabR2, outP, idxb, jb, R, OB, isems, gsems, osems):
        wid = lax.axis_index("s") * NC + lax.axis_index("c")
        b0 = wid * SUB
        iota = lax.iota(_I32, 16)

        def fire_idx(t, b):
            pltpu.async_copy(idxT.at[t, pl.ds(b0, SUB)], idxb[b], isems[b])

        def wait_idx(b):
            pltpu.make_async_copy(
                idxT.at[0, pl.ds(0, SUB)], idxb[b], isems[b]
            ).wait()

        def prep_and_fire_gather(b):
            for g in range(SUB // 16):
                ev = idxb[b][pl.ds(g * 16, 16)]
                jb[b][pl.ds(g * 16, 16)] = lax.shift_right_logical(ev, 1)
            pltpu.async_copy(tabR2.at[jb[b]], R[b], gsems[b])

        def wait_gather(b):
            pltpu.make_async_copy(
                tabR2.at[pl.ds(0, SUB)], R[b], gsems[b]
            ).wait()

        def fire_out(t, b):
            pltpu.async_copy(OB[b], outP.at[t, :, pl.ds(b0, SUB)], osems[b])

        def wait_out(b):
            pltpu.make_async_copy(
                OB[b], outP.at[0, :, pl.ds(0, SUB)], osems[b]
            ).wait()

        def transpose_block(b):
            def ibody(ib, carry):
                i0 = ib * 16
                rv = iota + _splat(i0)
                ev = idxb[b][pl.ds(i0, 16)]
                par64 = lax.shift_left(jnp.bitwise_and(ev, 1), 6)
                for cb in range(D // 16):
                    c0 = cb * 16
                    for d in range(16):
                        dia = jnp.bitwise_and(iota + d, 15)
                        cvec = dia + c0
                        vec = plsc.load_gather(R[b], [rv, par64 + cvec])
                        plsc.store_scatter(OB[b], [cvec, rv], vec)
                return carry

            lax.fori_loop(0, SUB // 16, ibody, 0)

        fire_idx(0, 0)
        wait_idx(0)
        prep_and_fire_gather(0)
        fire_idx(1, 1)

        def body(t2, carry):
            for b in range(2):
                t = t2 * 2 + b

                @pl.when(t < TT)
                def _():
                    wait_gather(b)

                    @pl.when(t + 1 < TT)
                    def _():
                        wait_idx(1 - b)
                        prep_and_fire_gather(1 - b)

                    @pl.when(t >= 2)
                    def _():
                        wait_out(b)

                    transpose_block(b)
                    fire_out(t, b)

                    @pl.when(t + 2 < TT)
                    def _():
                        fire_idx(t + 2, b)

            return carry

        lax.fori_loop(0, TT // 2, body, 0)
        wait_out(0)
        wait_out(1)

    tabT = table.T
    idxT = indices.T
    tabTail = lax.slice(tabT, (0, n_full * BE), (D, V))
    tabR = transpose_kernel(tabT, tabTail)
    tabR2 = tabR.reshape(V // 2, 2 * D)
    outP = gather_kernel(idxT, tabR2)
    return outP.transpose(2, 0, 1)


# speedup vs baseline: 2.2248x; 1.1288x over previous
import functools

import jax
import jax.numpy as jnp
from jax import lax
from jax.experimental import pallas as pl
from jax.experimental.pallas import tpu as pltpu
from jax.experimental.pallas import tpu_sc as plsc

_F32 = jnp.float32
_I32 = jnp.int32


def _splat(x):
    return jnp.full((16,), x, _I32)


def kernel(indices, table):
    B, H = indices.shape
    V, D = table.shape

    info = plsc.get_sparse_core_info()
    NC, NS = info.num_cores, info.num_subcores
    NW = NC * NS

    mesh = plsc.VectorSubcoreMesh(core_axis_name="c", subcore_axis_name="s")

    BE = 256
    n_full = V // BE
    tail = V - n_full * BE
    K = n_full // NW + 1
    KP = (K + 1) // 2 * 2

    @functools.partial(
        pl.kernel,
        mesh=mesh,
        out_type=jax.ShapeDtypeStruct((V * D,), _F32),
        scratch_types=[
            [pltpu.VMEM((D, BE), _F32)] * 2,
            [pltpu.VMEM((BE * D,), _F32)] * 2,
            pltpu.VMEM((D, tail), _F32),
            pltpu.VMEM((tail * D,), _F32),
            [pltpu.SemaphoreType.DMA] * 2,
            [pltpu.SemaphoreType.DMA] * 2,
        ],
        compiler_params=pltpu.CompilerParams(needs_layout_passes=False),
    )
    def transpose_kernel(tabT, tabTail, tabR, S, T, St, Tt, isems, osems):
        wid = lax.axis_index("s") * NC + lax.axis_index("c")
        iota = lax.iota(_I32, 16)

        def strip_of(k):
            return wid + k * NW

        def fire_in(k, b):
            s = strip_of(k)

            @pl.when(s < n_full)
            def _():
                pltpu.async_copy(tabT.at[:, pl.ds(s * BE, BE)], S[b], isems[b])

        def wait_in(b):
            pltpu.make_async_copy(tabT.at[:, pl.ds(0, BE)], S[b], isems[b]).wait()

        def fire_out(k, b):
            s = strip_of(k)
            pltpu.async_copy(T[b], tabR.at[pl.ds(s * BE * D, BE * D)], osems[b])

        def wait_out(b):
            pltpu.make_async_copy(T[b], tabR.at[pl.ds(0, BE * D)], osems[b]).wait()

        dcol = [jnp.bitwise_and(iota + d, 15) for d in range(16)]
        dcolD = [v * D for v in dcol]

        def transpose_strip(src, dst, width):
            def ibody(ib, carry):
                i0 = ib * 16
                iv = _splat(i0)
                for cb in range(D // 16):
                    c0 = cb * 16
                    cvec = iota + c0
                    sbase = iota + _splat(i0 * D + c0)
                    for d in range(16):
                        vec = plsc.load_gather(src, [cvec, iv + dcol[d]])
                        plsc.store_scatter(dst, [sbase + dcolD[d]], vec)
                return carry

            lax.fori_loop(0, width // 16, ibody, 0)

        fire_in(0, 0)

        def body(k2, carry):
            for b in range(2):
                k = k2 * 2 + b
                s = strip_of(k)

                @pl.when(s < n_full)
                def _():
                    wait_in(b)
                    fire_in(k + 1, 1 - b)

                    @pl.when(k >= 2)
                    def _():
                        wait_out(b)

                    transpose_strip(S[b], T[b], BE)
                    fire_out(k, b)

            return carry

        lax.fori_loop(0, KP // 2, body, 0)

        wait_out(0)
        wait_out(1)

        @pl.when(wid == NW - 1)
        def _():
            pltpu.sync_copy(tabTail, St)
            transpose_strip(St, Tt, tail)
            pltpu.sync_copy(Tt, tabR.at[pl.ds(n_full * BE * D, tail * D)])

    SUB = 128
    TT = H

    @functools.partial(
        pl.kernel,
        mesh=mesh,
        out_type=jax.ShapeDtypeStruct((H, D, B), _F32),
        scratch_types=[
            [pltpu.VMEM((SUB,), _I32)] * 2,
            [pltpu.VMEM((SUB,), _I32)] * 2,
            [pltpu.VMEM((SUB, 2 * D), _F32)] * 2,
            [pltpu.VMEM((D, SUB), _F32)] * 2,
            [pltpu.SemaphoreType.DMA] * 2,
            [pltpu.SemaphoreType.DMA] * 2,
            [pltpu.SemaphoreType.DMA] * 2,
        ],
        compiler_params=pltpu.CompilerParams(needs_layout_passes=False),
    )
    def gather_kernel(idxT, tabR2, outP, idxb, jb, R, OB, isems, gsems, osems):
        wid = lax.axis_index("s") * NC + lax.axis_index("c")
        b0 = wid * SUB
        iota = lax.iota(_I32, 16)

        def fire_idx(t, b):
            pltpu.async_copy(idxT.at[t, pl.ds(b0, SUB)], idxb[b], isems[b])

        def wait_idx(b):
            pltpu.make_async_copy(
                idxT.at[0, pl.ds(0, SUB)], idxb[b], isems[b]
            ).wait()

        def prep_and_fire_gather(b):
            for g in range(SUB // 16):
                ev = idxb[b][pl.ds(g * 16, 16)]
                jb[b][pl.ds(g * 16, 16)] = lax.shift_right_logical(ev, 1)
            pltpu.async_copy(tabR2.at[jb[b]], R[b], gsems[b])

        def wait_gather(b):
            pltpu.make_async_copy(
                tabR2.at[pl.ds(0, SUB)], R[b], gsems[b]
            ).wait()

        def fire_out(t, b):
            pltpu.async_copy(OB[b], outP.at[t, :, pl.ds(b0, SUB)], osems[b])

        def wait_out(b):
            pltpu.make_async_copy(
                OB[b], outP.at[0, :, pl.ds(0, SUB)], osems[b]
            ).wait()

        dcol = [jnp.bitwise_and(iota + d, 15) for d in range(16)]

        def transpose_block(b):
            def ibody(ib, carry):
                i0 = ib * 16
                rv = iota + _splat(i0)
                ev = idxb[b][pl.ds(i0, 16)]
                par64 = lax.shift_left(jnp.bitwise_and(ev, 1), 6)
                for cb in range(D // 16):
                    c0 = cb * 16
                    pc = par64 + c0
                    for d in range(16):
                        vec = plsc.load_gather(R[b], [rv, pc + dcol[d]])
                        plsc.store_scatter(OB[b], [dcol[d] + c0, rv], vec)
                return carry

            lax.fori_loop(0, SUB // 16, ibody, 0)

        fire_idx(0, 0)
        wait_idx(0)
        prep_and_fire_gather(0)
        fire_idx(1, 1)

        def body(t2, carry):
            for b in range(2):
                t = t2 * 2 + b

                @pl.when(t < TT)
                def _():
                    wait_gather(b)

                    @pl.when(t + 1 < TT)
                    def _():
                        wait_idx(1 - b)
                        prep_and_fire_gather(1 - b)

                    @pl.when(t >= 2)
                    def _():
                        wait_out(b)

                    transpose_block(b)
                    fire_out(t, b)

                    @pl.when(t + 2 < TT)
                    def _():
                        fire_idx(t + 2, b)

            return carry

        lax.fori_loop(0, TT // 2, body, 0)
        wait_out(0)
        wait_out(1)

    tabT = table.T
    idxT = indices.T
    tabTail = lax.slice(tabT, (0, n_full * BE), (D, V))
    tabR = transpose_kernel(tabT, tabTail)
    tabR2 = tabR.reshape(V // 2, 2 * D)
    outP = gather_kernel(idxT, tabR2)
    return outP.transpose(2, 0, 1)
